# baseline (speedup 1.0000x reference)
import functools
import math

import jax
import jax.numpy as jnp
from jax import lax
from jax.experimental import pallas as pl
from jax.experimental.pallas import tpu as pltpu
from jax.experimental.pallas import tpu_sc as plsc

NC = 2
NS = 16
L = 16
NW = NC * NS


def _mesh():
    return plsc.VectorSubcoreMesh(
        core_axis_name="c", subcore_axis_name="s",
        num_cores=NC, num_subcores=NS)


def _wid():
    return lax.axis_index("s") * NC + lax.axis_index("c")


def _iota16():
    return lax.iota(jnp.int32, 16)


def _transpose_block(src, dst, ncols, scale):
    iota = _iota16()
    jbs = [16 * jb + iota for jb in range(4)]

    @plsc.parallel_loop(0, 16, unroll=2)
    def _k(k):
        perm = jnp.bitwise_and(iota + k, 15)

        def loads(vb):
            batch = []
            for v in range(vb, vb + 4):
                v2v = perm + 16 * v
                rv = lax.shift_right_logical(v2v, 1)
                pbit = lax.shift_left(jnp.bitwise_and(v2v, 1), 6)
                for jb in range(4):
                    val = plsc.load_gather(src, [jbs[jb], v2v])
                    if scale is not None:
                        val = val * scale
                    batch.append((rv, jbs[jb] + pbit, val))
            return batch

        prev = loads(0)
        for vb in range(4, ncols // 16, 4):
            cur = loads(vb)
            for rv, cv, val in prev:
                plsc.store_scatter(dst, [rv, cv], val)
            prev = cur
        for rv, cv, val in prev:
            plsc.store_scatter(dst, [rv, cv], val)


BLK = 256


@functools.lru_cache(maxsize=None)
def _phase1(vocab: int, d: int):
    assert d == 64
    nblk = vocab // BLK
    tail = vocab % BLK
    assert tail in (0, 64)
    nfull = nblk // NW
    extra = nblk % NW
    scale = math.sqrt(d)

    def body(tt, tp, inb, outb, int_, outt, *sems):
        gs, ss = sems[:2], sems[2:]
        wid = _wid()

        def in_copies(t, b):
            v0 = (wid + NW * t) * BLK
            return [
                pltpu.make_async_copy(
                    tt.at[pl.ds(8 * tj, 8), pl.ds(v0, BLK)],
                    inb.at[b, pl.ds(8 * tj, 8)], gs[b])
                for tj in range(8)
            ]

        def fire_in(t, b):
            for cp in in_copies(t, b):
                cp.start()

        def wait_in(t, b):
            for cp in in_copies(t, b):
                cp.wait()

        def fire_out(t, b):
            return pltpu.make_async_copy(
                outb.at[b],
                tp.at[pl.ds((wid + NW * t) * (BLK // 2), BLK // 2)], ss[b])

        fire_in(0, 0)

        @pl.loop(0, nfull // 2)
        def _grp(g):
            for i in range(2):
                t = 2 * g + i
                @pl.when(t + 1 < nfull)
                def _():
                    fire_in(t + 1, (i + 1) % 2)
                wait_in(t, i)
                @pl.when(t >= 2)
                def _():
                    fire_out(t - 2, i).wait()
                _transpose_block(inb.at[i], outb.at[i], BLK, None)
                fire_out(t, i).start()

        fire_out(nfull - 2, 0).wait()
        fire_out(nfull - 1, 1).wait()

        if extra:
            @pl.when(wid < extra)
            def _():
                fire_in(nfull, 0)
                wait_in(nfull, 0)
                _transpose_block(inb.at[0], outb.at[0], BLK, None)
                fire_out(nfull, 0).start()
                fire_out(nfull, 0).wait()

        if tail:
            @pl.when(wid == extra)
            def _():
                tcps = [
                    pltpu.make_async_copy(
                        tt.at[pl.ds(8 * tj, 8), pl.ds(nblk * BLK, tail)],
                        int_.at[pl.ds(8 * tj, 8)], gs[0])
                    for tj in range(8)
                ]
                for cp in tcps:
                    cp.start()
                for cp in tcps:
                    cp.wait()
                _transpose_block(int_, outt, tail, None)
                cp2 = pltpu.make_async_copy(
                    outt, tp.at[pl.ds(nblk * (BLK // 2), tail // 2)], ss[0])
                cp2.start()
                cp2.wait()

    return pl.kernel(
        body,
        out_type=jax.ShapeDtypeStruct((vocab // 2, 128), jnp.float32),
        mesh=_mesh(),
        scratch_types=[
            pltpu.VMEM((2, 64, BLK), jnp.float32),
            pltpu.VMEM((2, BLK // 2, 128), jnp.float32),
            pltpu.VMEM((64, 64), jnp.float32),
            pltpu.VMEM((32, 128), jnp.float32),
        ] + [pltpu.SemaphoreType.DMA] * 4,
        compiler_params=pltpu.CompilerParams(
            use_tc_tiling_on_sc=True, needs_layout_passes=False),
    )


NBUF = 4


@functools.lru_cache(maxsize=None)
def _phase2(b: int, s: int, vocab: int, d: int):
    assert d == 64 and b % NW == 0 and s % NBUF == 0
    bpw = b // NW
    assert bpw == 128
    scale = math.sqrt(d)

    def body(xt, tl, out, xblk, gbuf, tbuf, *sems):
        gsems, ssems = sems[:NBUF], sems[NBUF:]
        wid = _wid()
        b0 = wid * bpw

        pltpu.sync_copy(xt.at[:, pl.ds(b0, bpw)], xblk)

        def gather(c, bi):
            return pltpu.make_async_copy(
                tl.at[xblk.at[c]], gbuf.at[bi], gsems[bi])

        def scatter(c, bi):
            return pltpu.make_async_copy(
                tbuf.at[bi], out.at[c, :, wid], ssems[bi])

        gather(0, 0).start()
        gather(1, 1).start()

        @pl.loop(0, s, step=NBUF)
        def _grp(g):
            for bi in range(NBUF):
                c = g + bi
                if bi >= 2:
                    scatter(c - 2, (bi + 2) % NBUF).wait()
                else:
                    @pl.when(c >= 2)
                    def _():
                        scatter(c - 2, (bi + 2) % NBUF).wait()
                @pl.when(c + 2 < s)
                def _():
                    gather(c + 2, (bi + 2) % NBUF).start()
                gather(c, bi).wait()
                src, dst = gbuf.at[bi], tbuf.at[bi]
                iota = _iota16()
                rows = [iota + 16 * g2 for g2 in range(bpw // L)]

                @plsc.parallel_loop(0, 16, unroll=2)
                def _diag(k):
                    perm = jnp.bitwise_and(iota + k, 15)
                    for h in range(d // L):
                        jv = perm + 16 * h
                        jb = lax.shift_right_logical(jv, 3)
                        jr = jnp.bitwise_and(jv, 7)
                        vals = [
                            plsc.load_gather(src, [rows[g2], jv]) * scale
                            for g2 in range(bpw // L)
                        ]
                        for g2 in range(bpw // L):
                            plsc.store_scatter(
                                dst, [jb, jr, rows[g2]], vals[g2])

                scatter(c, bi).start()

        scatter(s - 2, (s - 2) % NBUF).wait()
        scatter(s - 1, (s - 1) % NBUF).wait()

    return pl.kernel(
        body,
        out_type=jax.ShapeDtypeStruct(
            (s, d // 8, b // 128, 8, 128), jnp.float32),
        mesh=_mesh(),
        scratch_types=[
            pltpu.VMEM((s, bpw), jnp.int32),
            pltpu.VMEM((NBUF, bpw, d), jnp.float32),
            pltpu.VMEM((NBUF, d // 8, 8, bpw), jnp.float32),
        ] + [pltpu.SemaphoreType.DMA] * (2 * NBUF),
        compiler_params=pltpu.CompilerParams(
            use_tc_tiling_on_sc=False, needs_layout_passes=False),
    )


def kernel(x, table):
    b, s = x.shape
    vocab, d = table.shape
    xt = x.astype(jnp.int32).T
    tt = table.T
    tp = _phase1(vocab, d)(tt)
    tl = tp.reshape(vocab, d)
    out5 = _phase2(b, s, vocab, d)(xt, tl)
    return jnp.transpose(out5, (2, 4, 0, 1, 3)).reshape(b, s, d)


# speedup vs baseline: 1.1831x; 1.1831x over previous
import functools
import math

import jax
import jax.numpy as jnp
from jax import lax
from jax.experimental import pallas as pl
from jax.experimental.pallas import tpu as pltpu
from jax.experimental.pallas import tpu_sc as plsc

NC = 2
NS = 16
L = 16
NW = NC * NS


def _mesh():
    return plsc.VectorSubcoreMesh(
        core_axis_name="c", subcore_axis_name="s",
        num_cores=NC, num_subcores=NS)


def _wid():
    return lax.axis_index("s") * NC + lax.axis_index("c")


def _iota16():
    return lax.iota(jnp.int32, 16)


def _transpose_block(src, dst, ncols, scale):
    iota = _iota16()
    jbs = [16 * jb + iota for jb in range(4)]

    @plsc.parallel_loop(0, 16, unroll=2)
    def _k(k):
        perm = jnp.bitwise_and(iota + k, 15)
        for vb in range(0, ncols // 16, 4):
            stores = []
            for v in range(vb, vb + 4):
                v2v = perm + 16 * v
                rv = lax.shift_right_logical(v2v, 1)
                pbit = lax.shift_left(jnp.bitwise_and(v2v, 1), 6)
                for jb in range(4):
                    val = plsc.load_gather(src, [jbs[jb], v2v])
                    if scale is not None:
                        val = val * scale
                    stores.append((rv, jbs[jb] + pbit, val))
            for rv, cv, val in stores:
                plsc.store_scatter(dst, [rv, cv], val)


BLK = 256


@functools.lru_cache(maxsize=None)
def _phase1(vocab: int, d: int):
    assert d == 64
    nblk = vocab // BLK
    tail = vocab % BLK
    assert tail in (0, 64)
    nfull = nblk // NW
    extra = nblk % NW

    def body(tt, tp, inb, outb, int_, outt, *sems):
        gs, ss = sems[:2], sems[2:]
        wid = _wid()

        def in_copies(t, b):
            v0 = (wid + NW * t) * BLK
            return [
                pltpu.make_async_copy(
                    tt.at[pl.ds(8 * tj, 8), pl.ds(v0, BLK)],
                    inb.at[b, pl.ds(8 * tj, 8)], gs[b])
                for tj in range(8)
            ]

        def fire_in(t, b):
            for cp in in_copies(t, b):
                cp.start()

        def wait_in(t, b):
            for cp in in_copies(t, b):
                cp.wait()

        def fire_out(t, b):
            return pltpu.make_async_copy(
                outb.at[b],
                tp.at[pl.ds((wid + NW * t) * (BLK // 2), BLK // 2)], ss[b])

        fire_in(0, 0)

        @pl.loop(0, nfull // 2)
        def _grp(g):
            for i in range(2):
                t = 2 * g + i
                @pl.when(t + 1 < nfull)
                def _():
                    fire_in(t + 1, (i + 1) % 2)
                wait_in(t, i)
                @pl.when(t >= 2)
                def _():
                    fire_out(t - 2, i).wait()
                _transpose_block(inb.at[i], outb.at[i], BLK, None)
                fire_out(t, i).start()

        fire_out(nfull - 2, 0).wait()
        fire_out(nfull - 1, 1).wait()

        if extra:
            @pl.when(wid < extra)
            def _():
                fire_in(nfull, 0)
                wait_in(nfull, 0)
                _transpose_block(inb.at[0], outb.at[0], BLK, None)
                fire_out(nfull, 0).start()
                fire_out(nfull, 0).wait()

        if tail:
            @pl.when(wid == extra)
            def _():
                tcps = [
                    pltpu.make_async_copy(
                        tt.at[pl.ds(8 * tj, 8), pl.ds(nblk * BLK, tail)],
                        int_.at[pl.ds(8 * tj, 8)], gs[0])
                    for tj in range(8)
                ]
                for cp in tcps:
                    cp.start()
                for cp in tcps:
                    cp.wait()
                _transpose_block(int_, outt, tail, None)
                cp2 = pltpu.make_async_copy(
                    outt, tp.at[pl.ds(nblk * (BLK // 2), tail // 2)], ss[0])
                cp2.start()
                cp2.wait()

    return pl.kernel(
        body,
        out_type=jax.ShapeDtypeStruct((vocab // 2, 128), jnp.float32),
        mesh=_mesh(),
        scratch_types=[
            pltpu.VMEM((2, 64, BLK), jnp.float32),
            pltpu.VMEM((2, BLK // 2, 128), jnp.float32),
            pltpu.VMEM((64, 64), jnp.float32),
            pltpu.VMEM((32, 128), jnp.float32),
        ] + [pltpu.SemaphoreType.DMA] * 4,
        compiler_params=pltpu.CompilerParams(
            use_tc_tiling_on_sc=True, needs_layout_passes=False),
    )


NBUF = 4


@functools.lru_cache(maxsize=None)
def _phase2(b: int, s: int, vocab: int, d: int):
    assert d == 64 and b % NW == 0 and s % NBUF == 0
    bpw = b // NW
    assert bpw == 128
    scale = math.sqrt(d)

    def body(xt, tl, out, xblk, gbuf, tbuf, *sems):
        gsems, ssems = sems[:NBUF], sems[NBUF:]
        wid = _wid()
        b0 = wid * bpw

        pltpu.sync_copy(xt.at[:, pl.ds(b0, bpw)], xblk)

        def gather(c, bi):
            return pltpu.make_async_copy(
                tl.at[xblk.at[c]], gbuf.at[bi], gsems[bi])

        def scatter(c, bi):
            return pltpu.make_async_copy(
                tbuf.at[bi], out.at[c, :, wid], ssems[bi])

        gather(0, 0).start()
        gather(1, 1).start()

        @pl.loop(0, s, step=NBUF)
        def _grp(g):
            for bi in range(NBUF):
                c = g + bi
                if bi >= 2:
                    scatter(c - 2, (bi + 2) % NBUF).wait()
                else:
                    @pl.when(c >= 2)
                    def _():
                        scatter(c - 2, (bi + 2) % NBUF).wait()
                @pl.when(c + 2 < s)
                def _():
                    gather(c + 2, (bi + 2) % NBUF).start()
                gather(c, bi).wait()
                src, dst = gbuf.at[bi], tbuf.at[bi]
                iota = _iota16()
                rows = [iota + 16 * g2 for g2 in range(bpw // L)]

                @plsc.parallel_loop(0, 16, unroll=2)
                def _diag(k):
                    perm = jnp.bitwise_and(iota + k, 15)
                    for h in range(d // L):
                        jv = perm + 16 * h
                        jb = lax.shift_right_logical(jv, 3)
                        jr = jnp.bitwise_and(jv, 7)
                        vals = [
                            plsc.load_gather(src, [rows[g2], jv]) * scale
                            for g2 in range(bpw // L)
                        ]
                        for g2 in range(bpw // L):
                            plsc.store_scatter(
                                dst, [jb, jr, rows[g2]], vals[g2])

                scatter(c, bi).start()

        scatter(s - 2, (s - 2) % NBUF).wait()
        scatter(s - 1, (s - 1) % NBUF).wait()

    return pl.kernel(
        body,
        out_type=jax.ShapeDtypeStruct(
            (s, d // 8, b // 128, 8, 128), jnp.float32),
        mesh=_mesh(),
        scratch_types=[
            pltpu.VMEM((s, bpw), jnp.int32),
            pltpu.VMEM((NBUF, bpw, d), jnp.float32),
            pltpu.VMEM((NBUF, d // 8, 8, bpw), jnp.float32),
        ] + [pltpu.SemaphoreType.DMA] * (2 * NBUF),
        compiler_params=pltpu.CompilerParams(
            use_tc_tiling_on_sc=False, needs_layout_passes=False),
    )


def kernel(x, table):
    b, s = x.shape
    vocab, d = table.shape
    xt = x.astype(jnp.int32).T
    tt = table.T
    tp = _phase1(vocab, d)(tt)
    tl = tp.reshape(vocab, d)
    out5 = _phase2(b, s, vocab, d)(xt, tl)
    return jnp.transpose(out5, (2, 4, 0, 1, 3)).reshape(b, s, d)
